# baseline (speedup 1.0000x reference)
import functools
import jax
import jax.numpy as jnp
from jax import lax
from jax.experimental import pallas as pl
from jax.experimental.pallas import tpu as pltpu, tpu_sc as plsc

_N = 10000
_E = 160000
_NG = 32
_K = 9
_NW = 32
_NPW = 320
_OFF_TILE = 344
_OFF_PAD = _NW * _NPW + _OFF_TILE
_MPAD3 = 4 * _E + 12 * _N
_MPAD1 = _E + 15 * _N

def _make_agg_kernel(n_rows, cout, cpad, mpad):
    @functools.partial(
        pl.kernel,
        out_type=jax.ShapeDtypeStruct((_N * cout,), jnp.float32),
        mesh=plsc.VectorSubcoreMesh(core_axis_name="c", subcore_axis_name="s"),
        scratch_types=[
            pltpu.VMEM((_OFF_TILE,), jnp.int32),
            pltpu.VMEM((16,), jnp.int32),
            pltpu.VMEM((16,), jnp.float32),
            pltpu.VMEM((16, cpad), jnp.float32),
            pltpu.VMEM((cout,), jnp.float32),
            pltpu.SemaphoreType.DMA,
        ],
    )
    def k(xw_hbm, midx_hbm, mw_hbm, offp_hbm, out_hbm,
          off_v, idx_v, w_v, rows_v, acc_v, sem):
        wid = lax.axis_index("s") * 2 + lax.axis_index("c")
        n0 = wid * _NPW
        pltpu.sync_copy(offp_hbm.at[pl.ds(n0, _OFF_TILE)], off_v)
        n_cnt = jnp.minimum(_NPW, _N - n0)

        def node_body(i, carry):
            ov = off_v[pl.ds(i, 16)]
            c0 = ov[0] // 16
            c1 = ov[1] // 16

            def zero_body(s, carry0):
                acc_v[pl.ds(s * 16, 16)] = jnp.zeros((16,), jnp.float32)
                return carry0

            lax.fori_loop(0, cout // 16, zero_body, 0)

            def chunk_body(c, carry2):
                pltpu.sync_copy(midx_hbm.at[pl.ds(c * 16, 16)], idx_v)
                pltpu.sync_copy(mw_hbm.at[pl.ds(c * 16, 16)], w_v)
                pltpu.async_copy(xw_hbm.at[idx_v], rows_v, sem).wait()
                wv = w_v[...]
                wb = [jnp.full((16,), wv[j]) for j in range(16)]

                def s_body(s, carry3):
                    v = acc_v[pl.ds(s * 16, 16)]
                    for j in range(16):
                        v = v + wb[j] * rows_v[j, pl.ds(s * 16, 16)]
                    acc_v[pl.ds(s * 16, 16)] = v
                    return carry3

                lax.fori_loop(0, cout // 16, s_body, 0)
                return carry2

            lax.fori_loop(c0, c1, chunk_body, 0)
            pltpu.sync_copy(acc_v, out_hbm.at[pl.ds((n0 + i) * cout, cout)])
            return carry

        lax.fori_loop(0, n_cnt, node_body, 0)

    return k


def _mm_kernel(a_ref, b_ref, o_ref):
    o_ref[...] = jnp.dot(a_ref[...], b_ref[...],
                         preferred_element_type=jnp.float32)


def _tc_matmul(a, b):
    n, cin = a.shape
    d = b.shape[1]
    blk = 80
    return pl.pallas_call(
        _mm_kernel,
        grid=(n // blk,),
        in_specs=[
            pl.BlockSpec((blk, cin), lambda i: (i, 0)),
            pl.BlockSpec((cin, d), lambda i: (0, 0)),
        ],
        out_specs=pl.BlockSpec((blk, d), lambda i: (i, 0)),
        out_shape=jax.ShapeDtypeStruct((n, d), jnp.float32),
    )(a, b)


def _fc_kernel(pooled_ref, w_ref, b_ref, o_ref):
    o_ref[...] = jnp.dot(pooled_ref[...], w_ref[...],
                         preferred_element_type=jnp.float32) + b_ref[...]


def _spline_basis(edge_attr, ksz):
    p = edge_attr * (ksz - 1)
    i0 = jnp.clip(jnp.floor(p), 0, ksz - 2).astype(jnp.int32)
    f = p - i0.astype(p.dtype)
    idxs, ws = [], []
    for a in (0, 1):
        wa = f[:, 0] if a else (1.0 - f[:, 0])
        for b in (0, 1):
            wb = f[:, 1] if b else (1.0 - f[:, 1])
            idxs.append((i0[:, 0] + a) * ksz + (i0[:, 1] + b))
            ws.append(wa * wb)
    return jnp.stack(idxs, 1), jnp.stack(ws, 1)


def _build_messages(s_src, s_dst, s_attr, off1):
    deg = off1[1:] - off1[:-1]

    idx4, w4 = _spline_basis(s_attr, 3)
    cnt3p = ((4 * deg + 15) // 16) * 16
    offp3 = jnp.concatenate([jnp.zeros((1,), jnp.int32),
                             jnp.cumsum(cnt3p).astype(jnp.int32)])
    j = jnp.arange(4 * _E, dtype=jnp.int32)
    e = j // 4
    node = s_dst[e]
    pos3 = offp3[node] + (j - 4 * off1[node])
    midx3 = jnp.zeros((_MPAD3,), jnp.int32).at[pos3].set(
        s_src[e] * _K + idx4.reshape(-1))
    mw3 = jnp.zeros((_MPAD3,), jnp.float32).at[pos3].set(w4.reshape(-1))
    offp3p = jnp.pad(offp3, (0, _OFF_PAD - (_N + 1)), mode='edge')

    cnt1p = ((deg + 15) // 16) * 16
    offp1 = jnp.concatenate([jnp.zeros((1,), jnp.int32),
                             jnp.cumsum(cnt1p).astype(jnp.int32)])
    e1 = jnp.arange(_E, dtype=jnp.int32)
    pos1 = offp1[s_dst] + (e1 - off1[s_dst])
    midx1 = jnp.zeros((_MPAD1,), jnp.int32).at[pos1].set(s_src)
    mw1 = jnp.zeros((_MPAD1,), jnp.float32).at[pos1].set(1.0)
    offp1p = jnp.pad(offp1, (0, _OFF_PAD - (_N + 1)), mode='edge')

    return (midx3, mw3, offp3p), (midx1, mw1, offp1p), deg


def _spline_conv(x, p, ksz, msgs3, msgs1, inv_deg):
    W, R, b = p['W'], p['R'], p['b']
    K, cin, cout = W.shape
    bmat = jnp.concatenate([W.transpose(1, 0, 2).reshape(cin, K * cout), R], 1)
    xwr = _tc_matmul(x, bmat)
    cpad = max(cout, 128)
    xw = xwr[:, :K * cout].reshape(_N, K, cout)
    if cpad != cout:
        xw = jnp.pad(xw, ((0, 0), (0, 0), (0, cpad - cout)))
    xw = xw.reshape(_N * K, cpad)
    xr = xwr[:, K * cout:]
    midx, mw, offp = msgs3 if ksz == 3 else msgs1
    agg = _make_agg_kernel(xw.shape[0], cout, cpad, midx.shape[0])(
        xw, midx, mw, offp)
    return agg.reshape(_N, cout) * inv_deg[:, None] + xr + b


def _bn(x, p):
    m = jnp.mean(x, 0)
    v = jnp.var(x, 0)
    return (x - m) / jnp.sqrt(v + 1e-5) * p['g'] + p['b']


def _block(x, p, msgs3, msgs1, inv_deg):
    out = jax.nn.relu(_bn(_spline_conv(x, p['conv1'], 3, msgs3, msgs1, inv_deg), p['bn1']))
    out = _bn(_spline_conv(out, p['conv2'], 3, msgs3, msgs1, inv_deg), p['bn2'])
    res = x
    if 'ds' in p:
        res = _bn(_spline_conv(x, p['ds'], 1, msgs3, msgs1, inv_deg), p['ds_bn'])
    return jax.nn.relu(out + res)


def kernel(x, edge_index, edge_attr, pos, batch, params):
    src, dst = edge_index[0], edge_index[1]
    perm = jnp.argsort(dst)
    s_src = src[perm]
    s_dst = dst[perm]
    s_attr = edge_attr[perm]
    off1 = jnp.searchsorted(s_dst, jnp.arange(_N + 1, dtype=jnp.int32),
                            side='left').astype(jnp.int32)
    msgs3, msgs1, deg = _build_messages(s_src, s_dst, s_attr, off1)
    inv_deg = 1.0 / jnp.maximum(deg.astype(jnp.float32), 1.0)

    h = jax.nn.relu(_bn(_spline_conv(x, params['conv1'], 3, msgs3, msgs1, inv_deg),
                        params['bn1']))
    for name in ('layer1', 'layer2', 'layer3', 'layer4'):
        for bp in params[name]:
            h = _block(h, bp, msgs3, msgs1, inv_deg)

    v = jnp.clip(jnp.floor(pos / 4.0).astype(jnp.int32), 0, 1)
    cluster = batch.astype(jnp.int32) * 4 + v[:, 0] * 2 + v[:, 1]
    pooled = jax.ops.segment_max(h, cluster, num_segments=_NG * 4)
    pooled = jnp.where(jnp.isfinite(pooled), pooled, 0.0)
    out = pl.pallas_call(
        _fc_kernel,
        out_shape=jax.ShapeDtypeStruct((_NG * 4, 10), jnp.float32),
    )(pooled, params['fc']['W'], params['fc']['b'][None, :])
    return out
